# Initial kernel scaffold; baseline (speedup 1.0000x reference)
import jax
import jax.numpy as jnp
from jax.experimental import pallas as pl


def kernel(states, action, We1, be1, We2, be2, ge, bel, We3, be3, Wn1, bn1, Wn2, bn2, gn, bnl, Wn3, bn3):
    raise NotImplementedError("write your pallas kernel here")


# speedup vs baseline: 18.9166x; 18.9166x over previous
import jax
import jax.numpy as jnp
from jax.experimental import pallas as pl

B = 512
N = 32
D = 128
H = 128
A = 8
BB = 4


def _ln(x, g, b):
    m = jnp.mean(x, axis=-1, keepdims=True)
    v = jnp.mean((x - m) ** 2, axis=-1, keepdims=True)
    return (x - m) * jax.lax.rsqrt(v + 1e-5) * g + b


def _fused(node_ref, av_ref,
           We1a_ref, We1b_ref, be1_ref, We2_ref, be2_ref, ge_ref, bel_ref,
           We3_ref, be3_ref, Wn1n_ref, Wn1a_ref, Wn1g_ref, bn1_ref,
           Wn2_ref, bn2_ref, gn_ref, bnl_ref, Wn3_ref, bn3_ref, out_ref):
    f32 = jnp.float32
    node = node_ref[...].reshape(BB * N, D)
    u = jnp.dot(node, We1a_ref[...], preferred_element_type=f32)
    v = jnp.dot(node, We1b_ref[...], preferred_element_type=f32)
    e1 = jnp.maximum(
        u.reshape(BB, N, 1, H) + v.reshape(BB, 1, N, H)
        + be1_ref[...].reshape(1, 1, 1, H), 0.0)
    e1 = e1.reshape(BB * N * N, H)
    t = jnp.dot(e1, We2_ref[...], preferred_element_type=f32) + be2_ref[...]
    t = jnp.maximum(_ln(t, ge_ref[...], bel_ref[...]), 0.0)
    e3 = jnp.dot(t, We3_ref[...], preferred_element_type=f32) + be3_ref[...]
    e3 = e3.reshape(BB, N, N, H)
    ii = jax.lax.broadcasted_iota(jnp.int32, (1, N, N, 1), 1)
    jj = jax.lax.broadcasted_iota(jnp.int32, (1, N, N, 1), 2)
    mask = (ii != jj).astype(f32)
    agg = jnp.sum(e3 * mask, axis=2).reshape(BB * N, H)
    h = (jnp.dot(node, Wn1n_ref[...], preferred_element_type=f32)
         + jnp.dot(av_ref[...], Wn1a_ref[...], preferred_element_type=f32)
         + jnp.dot(agg, Wn1g_ref[...], preferred_element_type=f32)
         + bn1_ref[...])
    h = jnp.maximum(h, 0.0)
    t2 = jnp.dot(h, Wn2_ref[...], preferred_element_type=f32) + bn2_ref[...]
    t2 = jnp.maximum(_ln(t2, gn_ref[...], bnl_ref[...]), 0.0)
    out = jnp.dot(t2, Wn3_ref[...], preferred_element_type=f32) + bn3_ref[...]
    out_ref[...] = out.reshape(BB, N, D)


def kernel(states, action, We1, be1, We2, be2, ge, bel, We3, be3,
           Wn1, bn1, Wn2, bn2, gn, bnl, Wn3, bn3, interpret=False):
    av = jax.nn.one_hot(action, A * N, dtype=jnp.float32).reshape(B * N, A)
    We1a, We1b = We1[:D], We1[D:]
    Wn1n, Wn1a, Wn1g = Wn1[:D], Wn1[D : D + A], Wn1[D + A :]
    row = lambda x: x.reshape(1, -1)

    full = lambda shp: pl.BlockSpec(shp, lambda i: (0,) * len(shp))
    in_specs = [
        pl.BlockSpec((BB, N, D), lambda i: (i, 0, 0)),
        pl.BlockSpec((BB * N, A), lambda i: (i, 0)),
        full((D, H)), full((D, H)), full((1, H)),
        full((H, H)), full((1, H)), full((1, H)), full((1, H)),
        full((H, H)), full((1, H)),
        full((D, H)), full((A, H)), full((H, H)), full((1, H)),
        full((H, H)), full((1, H)), full((1, H)), full((1, H)),
        full((H, D)), full((1, D)),
    ]
    out = pl.pallas_call(
        _fused,
        grid=(B // BB,),
        in_specs=in_specs,
        out_specs=pl.BlockSpec((BB, N, D), lambda i: (i, 0, 0)),
        out_shape=jax.ShapeDtypeStruct((B, N, D), jnp.float32),
        interpret=interpret,
    )(states, av, We1a, We1b, row(be1), We2, row(be2), row(ge), row(bel),
      We3, row(be3), Wn1n, Wn1a, Wn1g, row(bn1), Wn2, row(bn2), row(gn),
      row(bnl), Wn3, row(bn3))
    return out


# speedup vs baseline: 22.7742x; 1.2039x over previous
import jax
import jax.numpy as jnp
from jax.experimental import pallas as pl

B = 512
N = 32
D = 128
H = 128
A = 8
BB = 8


def _ln(x, g, b):
    m = jnp.mean(x, axis=-1, keepdims=True)
    v = jnp.mean((x - m) ** 2, axis=-1, keepdims=True)
    return (x - m) * jax.lax.rsqrt(v + 1e-5) * g + b


def _fused(node_ref, av_ref,
           We1a_ref, We1b_ref, be1_ref, We2_ref, be2_ref, ge_ref, bel_ref,
           We3_ref, be3_ref, Wn1n_ref, Wn1a_ref, Wn1g_ref, bn1_ref,
           Wn2_ref, bn2_ref, gn_ref, bnl_ref, Wn3_ref, bn3_ref, out_ref):
    f32 = jnp.float32
    node = node_ref[...].reshape(BB * N, D)
    u = jnp.dot(node, We1a_ref[...], preferred_element_type=f32)
    v = jnp.dot(node, We1b_ref[...], preferred_element_type=f32)
    e1 = jnp.maximum(
        u.reshape(BB, N, 1, H) + v.reshape(BB, 1, N, H)
        + be1_ref[...].reshape(1, 1, 1, H), 0.0)
    e1 = e1.reshape(BB * N * N, H)
    t = jnp.dot(e1, We2_ref[...], preferred_element_type=f32) + be2_ref[...]
    t = jnp.maximum(_ln(t, ge_ref[...], bel_ref[...]), 0.0)
    e3 = jnp.dot(t, We3_ref[...], preferred_element_type=f32) + be3_ref[...]
    e3 = e3.reshape(BB, N, N, H)
    ii = jax.lax.broadcasted_iota(jnp.int32, (1, N, N, 1), 1)
    jj = jax.lax.broadcasted_iota(jnp.int32, (1, N, N, 1), 2)
    mask = (ii != jj).astype(f32)
    agg = jnp.sum(e3 * mask, axis=2).reshape(BB * N, H)
    h = (jnp.dot(node, Wn1n_ref[...], preferred_element_type=f32)
         + jnp.dot(av_ref[...], Wn1a_ref[...], preferred_element_type=f32)
         + jnp.dot(agg, Wn1g_ref[...], preferred_element_type=f32)
         + bn1_ref[...])
    h = jnp.maximum(h, 0.0)
    t2 = jnp.dot(h, Wn2_ref[...], preferred_element_type=f32) + bn2_ref[...]
    t2 = jnp.maximum(_ln(t2, gn_ref[...], bnl_ref[...]), 0.0)
    out = jnp.dot(t2, Wn3_ref[...], preferred_element_type=f32) + bn3_ref[...]
    out_ref[...] = out.reshape(BB, N, D)


def kernel(states, action, We1, be1, We2, be2, ge, bel, We3, be3,
           Wn1, bn1, Wn2, bn2, gn, bnl, Wn3, bn3, interpret=False):
    av = jax.nn.one_hot(action, A * N, dtype=jnp.float32).reshape(B * N, A)
    We1a, We1b = We1[:D], We1[D:]
    Wn1n, Wn1a, Wn1g = Wn1[:D], Wn1[D : D + A], Wn1[D + A :]
    row = lambda x: x.reshape(1, -1)

    full = lambda shp: pl.BlockSpec(shp, lambda i: (0,) * len(shp))
    in_specs = [
        pl.BlockSpec((BB, N, D), lambda i: (i, 0, 0)),
        pl.BlockSpec((BB * N, A), lambda i: (i, 0)),
        full((D, H)), full((D, H)), full((1, H)),
        full((H, H)), full((1, H)), full((1, H)), full((1, H)),
        full((H, H)), full((1, H)),
        full((D, H)), full((A, H)), full((H, H)), full((1, H)),
        full((H, H)), full((1, H)), full((1, H)), full((1, H)),
        full((H, D)), full((1, D)),
    ]
    out = pl.pallas_call(
        _fused,
        grid=(B // BB,),
        in_specs=in_specs,
        out_specs=pl.BlockSpec((BB, N, D), lambda i: (i, 0, 0)),
        out_shape=jax.ShapeDtypeStruct((B, N, D), jnp.float32),
        interpret=interpret,
    )(states, av, We1a, We1b, row(be1), We2, row(be2), row(ge), row(bel),
      We3, row(be3), Wn1n, Wn1a, Wn1g, row(bn1), Wn2, row(bn2), row(gn),
      row(bnl), Wn3, row(bn3))
    return out


# speedup vs baseline: 24.0604x; 1.0565x over previous
import jax
import jax.numpy as jnp
from jax.experimental import pallas as pl

B = 512
N = 32
D = 128
H = 128
A = 8
BB = 16


def _ln(x, g, b):
    m = jnp.mean(x, axis=-1, keepdims=True)
    v = jnp.mean((x - m) ** 2, axis=-1, keepdims=True)
    return (x - m) * jax.lax.rsqrt(v + 1e-5) * g + b


def _fused(node_ref, av_ref,
           We1a_ref, We1b_ref, be1_ref, We2_ref, be2_ref, ge_ref, bel_ref,
           We3_ref, be3_ref, Wn1n_ref, Wn1a_ref, Wn1g_ref, bn1_ref,
           Wn2_ref, bn2_ref, gn_ref, bnl_ref, Wn3_ref, bn3_ref, out_ref):
    f32 = jnp.float32
    node = node_ref[...].reshape(BB * N, D)
    u = jnp.dot(node, We1a_ref[...], preferred_element_type=f32)
    v = jnp.dot(node, We1b_ref[...], preferred_element_type=f32)
    e1 = jnp.maximum(
        u.reshape(BB, N, 1, H) + v.reshape(BB, 1, N, H)
        + be1_ref[...].reshape(1, 1, 1, H), 0.0)
    e1 = e1.reshape(BB * N * N, H)
    t = jnp.dot(e1, We2_ref[...], preferred_element_type=f32) + be2_ref[...]
    t = jnp.maximum(_ln(t, ge_ref[...], bel_ref[...]), 0.0)
    e3 = jnp.dot(t, We3_ref[...], preferred_element_type=f32) + be3_ref[...]
    e3 = e3.reshape(BB, N, N, H)
    ii = jax.lax.broadcasted_iota(jnp.int32, (1, N, N, 1), 1)
    jj = jax.lax.broadcasted_iota(jnp.int32, (1, N, N, 1), 2)
    mask = (ii != jj).astype(f32)
    agg = jnp.sum(e3 * mask, axis=2).reshape(BB * N, H)
    h = (jnp.dot(node, Wn1n_ref[...], preferred_element_type=f32)
         + jnp.dot(av_ref[...], Wn1a_ref[...], preferred_element_type=f32)
         + jnp.dot(agg, Wn1g_ref[...], preferred_element_type=f32)
         + bn1_ref[...])
    h = jnp.maximum(h, 0.0)
    t2 = jnp.dot(h, Wn2_ref[...], preferred_element_type=f32) + bn2_ref[...]
    t2 = jnp.maximum(_ln(t2, gn_ref[...], bnl_ref[...]), 0.0)
    out = jnp.dot(t2, Wn3_ref[...], preferred_element_type=f32) + bn3_ref[...]
    out_ref[...] = out.reshape(BB, N, D)


def kernel(states, action, We1, be1, We2, be2, ge, bel, We3, be3,
           Wn1, bn1, Wn2, bn2, gn, bnl, Wn3, bn3, interpret=False):
    av = jax.nn.one_hot(action, A * N, dtype=jnp.float32).reshape(B * N, A)
    We1a, We1b = We1[:D], We1[D:]
    Wn1n, Wn1a, Wn1g = Wn1[:D], Wn1[D : D + A], Wn1[D + A :]
    row = lambda x: x.reshape(1, -1)

    full = lambda shp: pl.BlockSpec(shp, lambda i: (0,) * len(shp))
    in_specs = [
        pl.BlockSpec((BB, N, D), lambda i: (i, 0, 0)),
        pl.BlockSpec((BB * N, A), lambda i: (i, 0)),
        full((D, H)), full((D, H)), full((1, H)),
        full((H, H)), full((1, H)), full((1, H)), full((1, H)),
        full((H, H)), full((1, H)),
        full((D, H)), full((A, H)), full((H, H)), full((1, H)),
        full((H, H)), full((1, H)), full((1, H)), full((1, H)),
        full((H, D)), full((1, D)),
    ]
    out = pl.pallas_call(
        _fused,
        grid=(B // BB,),
        in_specs=in_specs,
        out_specs=pl.BlockSpec((BB, N, D), lambda i: (i, 0, 0)),
        out_shape=jax.ShapeDtypeStruct((B, N, D), jnp.float32),
        interpret=interpret,
    )(states, av, We1a, We1b, row(be1), We2, row(be2), row(ge), row(bel),
      We3, row(be3), Wn1n, Wn1a, Wn1g, row(bn1), Wn2, row(bn2), row(gn),
      row(bnl), Wn3, row(bn3))
    return out


# speedup vs baseline: 24.6280x; 1.0236x over previous
import jax
import jax.numpy as jnp
from jax.experimental import pallas as pl

B = 512
N = 32
D = 128
H = 128
A = 8
BB = 32


def _ln(x, g, b):
    m = jnp.mean(x, axis=-1, keepdims=True)
    v = jnp.mean((x - m) ** 2, axis=-1, keepdims=True)
    return (x - m) * jax.lax.rsqrt(v + 1e-5) * g + b


def _fused(node_ref, av_ref,
           We1a_ref, We1b_ref, be1_ref, We2_ref, be2_ref, ge_ref, bel_ref,
           We3_ref, be3_ref, Wn1n_ref, Wn1a_ref, Wn1g_ref, bn1_ref,
           Wn2_ref, bn2_ref, gn_ref, bnl_ref, Wn3_ref, bn3_ref, out_ref):
    f32 = jnp.float32
    node = node_ref[...].reshape(BB * N, D)
    u = jnp.dot(node, We1a_ref[...], preferred_element_type=f32)
    v = jnp.dot(node, We1b_ref[...], preferred_element_type=f32)
    e1 = jnp.maximum(
        u.reshape(BB, N, 1, H) + v.reshape(BB, 1, N, H)
        + be1_ref[...].reshape(1, 1, 1, H), 0.0)
    e1 = e1.reshape(BB * N * N, H)
    t = jnp.dot(e1, We2_ref[...], preferred_element_type=f32) + be2_ref[...]
    t = jnp.maximum(_ln(t, ge_ref[...], bel_ref[...]), 0.0)
    e3 = jnp.dot(t, We3_ref[...], preferred_element_type=f32) + be3_ref[...]
    e3 = e3.reshape(BB, N, N, H)
    ii = jax.lax.broadcasted_iota(jnp.int32, (1, N, N, 1), 1)
    jj = jax.lax.broadcasted_iota(jnp.int32, (1, N, N, 1), 2)
    mask = (ii != jj).astype(f32)
    agg = jnp.sum(e3 * mask, axis=2).reshape(BB * N, H)
    h = (jnp.dot(node, Wn1n_ref[...], preferred_element_type=f32)
         + jnp.dot(av_ref[...], Wn1a_ref[...], preferred_element_type=f32)
         + jnp.dot(agg, Wn1g_ref[...], preferred_element_type=f32)
         + bn1_ref[...])
    h = jnp.maximum(h, 0.0)
    t2 = jnp.dot(h, Wn2_ref[...], preferred_element_type=f32) + bn2_ref[...]
    t2 = jnp.maximum(_ln(t2, gn_ref[...], bnl_ref[...]), 0.0)
    out = jnp.dot(t2, Wn3_ref[...], preferred_element_type=f32) + bn3_ref[...]
    out_ref[...] = out.reshape(BB, N, D)


def kernel(states, action, We1, be1, We2, be2, ge, bel, We3, be3,
           Wn1, bn1, Wn2, bn2, gn, bnl, Wn3, bn3, interpret=False):
    av = jax.nn.one_hot(action, A * N, dtype=jnp.float32).reshape(B * N, A)
    We1a, We1b = We1[:D], We1[D:]
    Wn1n, Wn1a, Wn1g = Wn1[:D], Wn1[D : D + A], Wn1[D + A :]
    row = lambda x: x.reshape(1, -1)

    full = lambda shp: pl.BlockSpec(shp, lambda i: (0,) * len(shp))
    in_specs = [
        pl.BlockSpec((BB, N, D), lambda i: (i, 0, 0)),
        pl.BlockSpec((BB * N, A), lambda i: (i, 0)),
        full((D, H)), full((D, H)), full((1, H)),
        full((H, H)), full((1, H)), full((1, H)), full((1, H)),
        full((H, H)), full((1, H)),
        full((D, H)), full((A, H)), full((H, H)), full((1, H)),
        full((H, H)), full((1, H)), full((1, H)), full((1, H)),
        full((H, D)), full((1, D)),
    ]
    out = pl.pallas_call(
        _fused,
        grid=(B // BB,),
        in_specs=in_specs,
        out_specs=pl.BlockSpec((BB, N, D), lambda i: (i, 0, 0)),
        out_shape=jax.ShapeDtypeStruct((B, N, D), jnp.float32),
        interpret=interpret,
    )(states, av, We1a, We1b, row(be1), We2, row(be2), row(ge), row(bel),
      We3, row(be3), Wn1n, Wn1a, Wn1g, row(bn1), Wn2, row(bn2), row(gn),
      row(bnl), Wn3, row(bn3))
    return out
